# baseline (speedup 1.0000x reference)
import functools

import jax
import jax.numpy as jnp
from jax import lax
from jax.experimental import pallas as pl
from jax.experimental.pallas import tpu as pltpu
from jax.experimental.pallas import tpu_sc as plsc

N = 10000
E = 320000
DIN = 128
H1 = 8
F1 = 16
D1 = H1 * F1
NCLS = 40

RA1 = 144
RA2 = 48

NSC = 2
NTEC = 16
NW = NSC * NTEC
NPAD = 10240
ROWS = NPAD // NTEC
CB = 80
EPW = NPAD
E2 = NW * EPW
NBLK = EPW // CB

BLK = 512
GRID = NPAD // BLK
BLK5 = 400
GRID5 = N // BLK5

_f32 = jnp.float32


def _bcast_lane(v, k):
    idx = jnp.full((16, 1), k, jnp.int32)
    dn = lax.GatherDimensionNumbers(
        offset_dims=(), collapsed_slice_dims=(0,), start_index_map=(0,))
    return lax.gather(v, idx, dn, (1,),
                      mode=lax.GatherScatterMode.PROMISE_IN_BOUNDS)


def _s1_body(x_ref, w_ref, as_ref, ad_ref, tabA_ref, tabB_ref, mx_ref):
    i = pl.program_id(0)
    h = jnp.dot(x_ref[...], w_ref[...], preferred_element_type=_f32)
    asrc = jnp.dot(h, as_ref[...], preferred_element_type=_f32)
    adst = jnp.dot(h, ad_ref[...], preferred_element_type=_f32)
    z = jnp.zeros((BLK, 8), _f32)
    tabA_ref[...] = jnp.concatenate([h, asrc, z], axis=1)
    tabB_ref[...] = jnp.concatenate([adst, z], axis=1)
    m = jnp.concatenate([jnp.max(asrc, axis=0, keepdims=True),
                         jnp.max(adst, axis=0, keepdims=True)], axis=1)

    @pl.when(i == 0)
    def _():
        mx_ref[...] = m

    @pl.when(i > 0)
    def _():
        mx_ref[...] = jnp.maximum(mx_ref[...], m)


def _stage1(x, W1, As, Ad):
    return pl.pallas_call(
        _s1_body,
        grid=(GRID,),
        in_specs=[
            pl.BlockSpec((BLK, DIN), lambda i: (i, 0)),
            pl.BlockSpec((DIN, D1), lambda i: (0, 0)),
            pl.BlockSpec((DIN, H1), lambda i: (0, 0)),
            pl.BlockSpec((DIN, H1), lambda i: (0, 0)),
        ],
        out_specs=[
            pl.BlockSpec((BLK, RA1), lambda i: (i, 0)),
            pl.BlockSpec((BLK, 16), lambda i: (i, 0)),
            pl.BlockSpec((1, 16), lambda i: (0, 0)),
        ],
        out_shape=[
            jax.ShapeDtypeStruct((NPAD, RA1), _f32),
            jax.ShapeDtypeStruct((NPAD, 16), _f32),
            jax.ShapeDtypeStruct((1, 16), _f32),
        ],
    )(x, W1, As, Ad)


def _edge_kernel_body(row_w, head_slots, tabA, tabB, epk, zrows, mvec, out,
                      ei0, ei1, ei2, ei3, rowA0, rowA1, rowB0, rowB1, mv, acc,
                      semI0, semI1, semI2, semI3,
                      semA0, semA1, semB0, semB1, semS0, semS1):
    c = lax.axis_index("c")
    s = lax.axis_index("s")
    w = c * NTEC + s
    gbase = w * NBLK
    pltpu.sync_copy(zrows, acc.at[pl.ds(s * ROWS, ROWS)])
    pltpu.sync_copy(mvec, mv)
    plsc.subcore_barrier()
    mvv = mv[...]
    asl = row_w - 16

    ei = (ei0, ei1, ei2, ei3)
    semI = (semI0, semI1, semI2, semI3)
    rowA = (rowA0, rowA1)
    rowB = (rowB0, rowB1)
    semA = (semA0, semA1)
    semB = (semB0, semB1)
    semS = (semS0, semS1)

    def idx_copy(b, j):
        return pltpu.make_async_copy(epk.at[gbase + b], ei[j], semI[j])

    def gatherA(b, p, j):
        return pltpu.make_async_copy(tabA.at[ei[j].at[0]], rowA[p], semA[p])

    def gatherB(b, p, j):
        return pltpu.make_async_copy(tabB.at[ei[j].at[1]], rowB[p], semB[p])

    def scatter(b, p, j):
        return pltpu.make_async_copy(rowA[p], acc.at[ei[j].at[1]], semS[p])

    def compute(p):
        ra_ref = rowA[p]
        rb_ref = rowB[p]

        @plsc.parallel_loop(0, CB, step=1, unroll=8)
        def edge(e):
            ra = ra_ref[e, pl.ds(asl, 16)]
            rb = rb_ref[e, pl.ds(0, 16)]
            sv = ra + rb
            al = jnp.where(sv > 0, sv, 0.2 * sv)
            p_ = jnp.exp(al - mvv)
            if head_slots == 8:
                ra_ref[e, pl.ds(asl, 16)] = p_
                for k in range(8):
                    pk = _bcast_lane(p_, k)
                    ra_ref[e, pl.ds(k * 16, 16)] = (
                        ra_ref[e, pl.ds(k * 16, 16)] * pk)
            else:
                pk = _bcast_lane(p_, 9)
                ra_ref[e, pl.ds(asl, 16)] = ra * pk
                for k in range((row_w - 16) // 16):
                    ra_ref[e, pl.ds(k * 16, 16)] = (
                        ra_ref[e, pl.ds(k * 16, 16)] * pk)

    idx_copy(0, 0).start()
    idx_copy(1, 1).start()
    idx_copy(0, 0).wait()
    gatherA(0, 0, 0).start()
    gatherB(0, 0, 0).start()

    def quad(i, carry):
        for u in range(4):
            b = 4 * i + u
            p = u % 2
            j = u
            jn = (u + 1) % 4
            jp = (u + 2) % 4
            jq = (u + 3) % 4

            @pl.when(b >= 1)
            def _(b=b, p=p, jq=jq):
                scatter(b - 1, 1 - p, jq).wait()

            @pl.when(b + 2 < NBLK)
            def _(b=b, jp=jp):
                idx_copy(b + 2, jp).start()

            @pl.when(b + 1 < NBLK)
            def _(b=b, p=p, jn=jn):
                idx_copy(b + 1, jn).wait()
                gatherA(b + 1, 1 - p, jn).start()
                gatherB(b + 1, 1 - p, jn).start()

            gatherA(b, p, j).wait()
            gatherB(b, p, j).wait()
            compute(p)
            pltpu.async_copy(rowA[p], acc.at[ei[j].at[1]], semS[p], add=True)
        return carry

    lax.fori_loop(0, NBLK // 4, quad, 0)
    scatter(NBLK - 1, 1, 3).wait()
    plsc.subcore_barrier()
    pltpu.sync_copy(acc.at[pl.ds(s * ROWS, ROWS)],
                    out.at[c, pl.ds(s * ROWS, ROWS)])


def _edge_pass(row_w, head_slots, tabA, tabB, epk, zrows, mvec):
    mesh = plsc.VectorSubcoreMesh(core_axis_name="c", subcore_axis_name="s",
                                  num_cores=NSC, num_subcores=NTEC)
    body = functools.partial(_edge_kernel_body, row_w, head_slots)
    return pl.kernel(
        body,
        out_type=jax.ShapeDtypeStruct((NSC, NPAD, row_w), _f32),
        mesh=mesh,
        scratch_types=[
            pltpu.VMEM((2, CB), jnp.int32),
            pltpu.VMEM((2, CB), jnp.int32),
            pltpu.VMEM((2, CB), jnp.int32),
            pltpu.VMEM((2, CB), jnp.int32),
            pltpu.VMEM((CB, row_w), _f32),
            pltpu.VMEM((CB, row_w), _f32),
            pltpu.VMEM((CB, 16), _f32),
            pltpu.VMEM((CB, 16), _f32),
            pltpu.VMEM((16,), _f32),
            pltpu.VMEM_SHARED((NPAD, row_w), _f32),
            pltpu.SemaphoreType.DMA,
            pltpu.SemaphoreType.DMA,
            pltpu.SemaphoreType.DMA,
            pltpu.SemaphoreType.DMA,
            pltpu.SemaphoreType.DMA,
            pltpu.SemaphoreType.DMA,
            pltpu.SemaphoreType.DMA,
            pltpu.SemaphoreType.DMA,
            pltpu.SemaphoreType.DMA,
            pltpu.SemaphoreType.DMA,
        ],
        compiler_params=pltpu.CompilerParams(use_tc_tiling_on_sc=False),
    )(tabA, tabB, epk, zrows, mvec)


def _s3_body(acc_ref, b1_ref, r_ref, w2_ref, s2_ref, d2_ref,
             tabA_ref, tabB_ref, mx_ref):
    i = pl.program_id(0)
    num = acc_ref[0] + acc_ref[1]
    den = num[:, D1:D1 + H1]
    dw = jnp.dot(den, r_ref[...], preferred_element_type=_f32)
    o1 = num[:, :D1] / (dw + 1e-16) + b1_ref[...]
    o1 = jnp.where(o1 > 0, o1, jnp.exp(o1) - 1.0)
    h2 = jnp.dot(o1, w2_ref[...], preferred_element_type=_f32)
    a2s = jnp.dot(h2, s2_ref[...], preferred_element_type=_f32)
    a2d = jnp.dot(h2, d2_ref[...], preferred_element_type=_f32)
    one = jnp.ones((BLK, 1), _f32)
    z6 = jnp.zeros((BLK, 6), _f32)
    z9 = jnp.zeros((BLK, 9), _f32)
    tabA_ref[...] = jnp.concatenate([h2, one, a2s, z6], axis=1)
    tabB_ref[...] = jnp.concatenate([z9, a2d, z6], axis=1)
    m = jnp.concatenate(
        [jnp.max(a2s, axis=0, keepdims=True),
         jnp.max(a2d, axis=0, keepdims=True),
         jnp.zeros((1, 14), _f32)], axis=1)

    @pl.when(i == 0)
    def _():
        mx_ref[...] = m

    @pl.when(i > 0)
    def _():
        mx_ref[...] = jnp.maximum(mx_ref[...], m)


def _stage3(acc1, b1, R, W2, s2, d2):
    return pl.pallas_call(
        _s3_body,
        grid=(GRID,),
        in_specs=[
            pl.BlockSpec((NSC, BLK, RA1), lambda i: (0, i, 0)),
            pl.BlockSpec((1, D1), lambda i: (0, 0)),
            pl.BlockSpec((H1, D1), lambda i: (0, 0)),
            pl.BlockSpec((D1, NCLS), lambda i: (0, 0)),
            pl.BlockSpec((NCLS, 1), lambda i: (0, 0)),
            pl.BlockSpec((NCLS, 1), lambda i: (0, 0)),
        ],
        out_specs=[
            pl.BlockSpec((BLK, RA2), lambda i: (i, 0)),
            pl.BlockSpec((BLK, 16), lambda i: (i, 0)),
            pl.BlockSpec((1, 16), lambda i: (0, 0)),
        ],
        out_shape=[
            jax.ShapeDtypeStruct((NPAD, RA2), _f32),
            jax.ShapeDtypeStruct((NPAD, 16), _f32),
            jax.ShapeDtypeStruct((1, 16), _f32),
        ],
    )(acc1, b1, R, W2, s2, d2)


def _s5_body(acc_ref, b2_ref, out_ref):
    num = acc_ref[0] + acc_ref[1]
    den = num[:, NCLS:NCLS + 1]
    lg = num[:, :NCLS] / (den + 1e-16) + b2_ref[...]
    m = jnp.max(lg, axis=1, keepdims=True)
    ls = lg - m
    out_ref[...] = ls - jnp.log(jnp.sum(jnp.exp(ls), axis=1, keepdims=True))


def _stage5(acc2, b2):
    return pl.pallas_call(
        _s5_body,
        grid=(GRID5,),
        in_specs=[
            pl.BlockSpec((NSC, BLK5, RA2), lambda i: (0, i, 0)),
            pl.BlockSpec((1, NCLS), lambda i: (0, 0)),
        ],
        out_specs=pl.BlockSpec((BLK5, NCLS), lambda i: (i, 0)),
        out_shape=jax.ShapeDtypeStruct((N, NCLS), _f32),
    )(acc2, b2)


def kernel(x, edge_index, W1, att_src1, att_dst1, b1, W2, att_src2, att_dst2,
           b2):
    edge_index = edge_index.astype(jnp.int32)
    epw_real = E // NW
    pad_per_w = EPW - epw_real
    ei = edge_index.reshape(2, NW, epw_real)
    padrow = (N + jnp.arange(pad_per_w, dtype=jnp.int32) % (NPAD - N))
    pad = jnp.broadcast_to(padrow, (2, NW, pad_per_w))
    epk = (jnp.concatenate([ei, pad], axis=2)
           .reshape(2, NW, NBLK, CB)
           .transpose(1, 2, 0, 3)
           .reshape(NW * NBLK, 2, CB))

    xp = jnp.concatenate([x, jnp.zeros((NPAD - N, DIN), _f32)], axis=0)

    eye = jnp.eye(H1, dtype=_f32)
    As = (att_src1[:, :, None] * eye[:, None, :]).reshape(D1, H1)
    Ad = (att_dst1[:, :, None] * eye[:, None, :]).reshape(D1, H1)
    R = jnp.repeat(eye, F1, axis=1)

    tabA1, tabB1, mx1 = _stage1(xp, W1, As, Ad)
    m1 = jnp.maximum(mx1[0, :H1] + mx1[0, H1:], 0.0)
    mvec1 = jnp.concatenate([m1, jnp.zeros((8,), _f32)])
    zrows1 = jnp.zeros((ROWS, RA1), _f32)

    acc1 = _edge_pass(RA1, H1, tabA1, tabB1, epk, zrows1, mvec1)

    tabA2, tabB2, mx2 = _stage3(acc1, b1.reshape(1, D1), R, W2,
                                att_src2.reshape(NCLS, 1),
                                att_dst2.reshape(NCLS, 1))
    m2 = jnp.maximum(mx2[0, 0] + mx2[0, 1], 0.0)
    mvec2 = jnp.zeros((16,), _f32).at[9].set(m2)
    zrows2 = jnp.zeros((ROWS, RA2), _f32)

    acc2 = _edge_pass(RA2, 1, tabA2, tabB2, epk, zrows2, mvec2)

    return _stage5(acc2, b2.reshape(1, NCLS))


# speedup vs baseline: 1.2653x; 1.2653x over previous
import functools

import jax
import jax.numpy as jnp
from jax import lax
from jax.experimental import pallas as pl
from jax.experimental.pallas import tpu as pltpu
from jax.experimental.pallas import tpu_sc as plsc

N = 10000
E = 320000
DIN = 128
H1 = 8
F1 = 16
D1 = H1 * F1
NCLS = 40

RA1 = 144
RA2 = 48

NSC = 2
NTEC = 16
NW = NSC * NTEC
NPAD = 10240
ROWS = NPAD // NTEC
CB = 80
EPW = NPAD
E2 = NW * EPW
NBLK = EPW // CB

BLK = 512
GRID = NPAD // BLK
BLK5 = 400
GRID5 = N // BLK5

_f32 = jnp.float32


def _bcast_lane(v, k):
    idx = jnp.full((16, 1), k, jnp.int32)
    dn = lax.GatherDimensionNumbers(
        offset_dims=(), collapsed_slice_dims=(0,), start_index_map=(0,))
    return lax.gather(v, idx, dn, (1,),
                      mode=lax.GatherScatterMode.PROMISE_IN_BOUNDS)


def _s1_body(x_ref, w_ref, as_ref, ad_ref, tabA_ref, tabB_ref, mx_ref):
    i = pl.program_id(0)
    h = jnp.dot(x_ref[...], w_ref[...], preferred_element_type=_f32)
    asrc = jnp.dot(h, as_ref[...], preferred_element_type=_f32)
    adst = jnp.dot(h, ad_ref[...], preferred_element_type=_f32)
    z = jnp.zeros((BLK, 8), _f32)
    tabA_ref[...] = jnp.concatenate([h, asrc, z], axis=1)
    tabB_ref[...] = jnp.concatenate([adst, z], axis=1)
    m = jnp.concatenate([jnp.max(asrc, axis=0, keepdims=True),
                         jnp.max(adst, axis=0, keepdims=True)], axis=1)

    @pl.when(i == 0)
    def _():
        mx_ref[...] = m

    @pl.when(i > 0)
    def _():
        mx_ref[...] = jnp.maximum(mx_ref[...], m)


def _stage1(x, W1, As, Ad):
    return pl.pallas_call(
        _s1_body,
        grid=(GRID,),
        in_specs=[
            pl.BlockSpec((BLK, DIN), lambda i: (i, 0)),
            pl.BlockSpec((DIN, D1), lambda i: (0, 0)),
            pl.BlockSpec((DIN, H1), lambda i: (0, 0)),
            pl.BlockSpec((DIN, H1), lambda i: (0, 0)),
        ],
        out_specs=[
            pl.BlockSpec((BLK, RA1), lambda i: (i, 0)),
            pl.BlockSpec((BLK, 16), lambda i: (i, 0)),
            pl.BlockSpec((1, 16), lambda i: (0, 0)),
        ],
        out_shape=[
            jax.ShapeDtypeStruct((NPAD, RA1), _f32),
            jax.ShapeDtypeStruct((NPAD, 16), _f32),
            jax.ShapeDtypeStruct((1, 16), _f32),
        ],
    )(x, W1, As, Ad)


def _edge_kernel_body(row_w, head_slots, tabA, tabB, epk, zrows, mvec, out,
                      ei0, ei1, ei2, ei3, rowA0, rowA1, rowB0, rowB1, mv, acc,
                      semI0, semI1, semI2, semI3,
                      semA0, semA1, semB0, semB1, semS0, semS1):
    c = lax.axis_index("c")
    s = lax.axis_index("s")
    w = c * NTEC + s
    gbase = w * NBLK
    pltpu.sync_copy(zrows, acc.at[pl.ds(s * ROWS, ROWS)])
    pltpu.sync_copy(mvec, mv)
    plsc.subcore_barrier()
    mvv = mv[...]
    asl = row_w - 16

    ei = (ei0, ei1, ei2, ei3)
    semI = (semI0, semI1, semI2, semI3)
    rowA = (rowA0, rowA1)
    rowB = (rowB0, rowB1)
    semA = (semA0, semA1)
    semB = (semB0, semB1)
    semS = (semS0, semS1)

    def idx_copy(b, j):
        return pltpu.make_async_copy(epk.at[gbase + b], ei[j], semI[j])

    def gatherA(b, p, j):
        return pltpu.make_async_copy(tabA.at[ei[j].at[0]], rowA[p], semA[p])

    def gatherB(b, p, j):
        return pltpu.make_async_copy(tabB.at[ei[j].at[1]], rowB[p], semB[p])

    def scatter(b, p, j):
        return pltpu.make_async_copy(rowA[p], acc.at[ei[j].at[1]], semS[p])

    def compute(p):
        ra_ref = rowA[p]
        rb_ref = rowB[p]

        @plsc.parallel_loop(0, CB, step=1, unroll=2)
        def edge(e):
            ra = ra_ref[e, pl.ds(asl, 16)]
            rb = rb_ref[e, pl.ds(0, 16)]
            sv = ra + rb
            al = jnp.where(sv > 0, sv, 0.2 * sv)
            p_ = jnp.exp(al - mvv)
            if head_slots == 8:
                ra_ref[e, pl.ds(asl, 16)] = p_
                for k in range(8):
                    pk = _bcast_lane(p_, k)
                    ra_ref[e, pl.ds(k * 16, 16)] = (
                        ra_ref[e, pl.ds(k * 16, 16)] * pk)
            else:
                pk = _bcast_lane(p_, 9)
                ra_ref[e, pl.ds(asl, 16)] = ra * pk
                for k in range((row_w - 16) // 16):
                    ra_ref[e, pl.ds(k * 16, 16)] = (
                        ra_ref[e, pl.ds(k * 16, 16)] * pk)

    idx_copy(0, 0).start()
    idx_copy(1, 1).start()
    idx_copy(0, 0).wait()
    gatherA(0, 0, 0).start()
    gatherB(0, 0, 0).start()

    def quad(i, carry):
        for u in range(4):
            b = 4 * i + u
            p = u % 2
            j = u
            jn = (u + 1) % 4
            jp = (u + 2) % 4
            jq = (u + 3) % 4

            @pl.when(b >= 1)
            def _(b=b, p=p, jq=jq):
                scatter(b - 1, 1 - p, jq).wait()

            @pl.when(b + 2 < NBLK)
            def _(b=b, jp=jp):
                idx_copy(b + 2, jp).start()

            @pl.when(b + 1 < NBLK)
            def _(b=b, p=p, jn=jn):
                idx_copy(b + 1, jn).wait()
                gatherA(b + 1, 1 - p, jn).start()
                gatherB(b + 1, 1 - p, jn).start()

            gatherA(b, p, j).wait()
            gatherB(b, p, j).wait()
            compute(p)
            pltpu.async_copy(rowA[p], acc.at[ei[j].at[1]], semS[p], add=True)
        return carry

    lax.fori_loop(0, NBLK // 4, quad, 0)
    scatter(NBLK - 1, 1, 3).wait()
    plsc.subcore_barrier()
    pltpu.sync_copy(acc.at[pl.ds(s * ROWS, ROWS)],
                    out.at[c, pl.ds(s * ROWS, ROWS)])


def _edge_pass(row_w, head_slots, tabA, tabB, epk, zrows, mvec):
    mesh = plsc.VectorSubcoreMesh(core_axis_name="c", subcore_axis_name="s",
                                  num_cores=NSC, num_subcores=NTEC)
    body = functools.partial(_edge_kernel_body, row_w, head_slots)
    return pl.kernel(
        body,
        out_type=jax.ShapeDtypeStruct((NSC, NPAD, row_w), _f32),
        mesh=mesh,
        scratch_types=[
            pltpu.VMEM((2, CB), jnp.int32),
            pltpu.VMEM((2, CB), jnp.int32),
            pltpu.VMEM((2, CB), jnp.int32),
            pltpu.VMEM((2, CB), jnp.int32),
            pltpu.VMEM((CB, row_w), _f32),
            pltpu.VMEM((CB, row_w), _f32),
            pltpu.VMEM((CB, 16), _f32),
            pltpu.VMEM((CB, 16), _f32),
            pltpu.VMEM((16,), _f32),
            pltpu.VMEM_SHARED((NPAD, row_w), _f32),
            pltpu.SemaphoreType.DMA,
            pltpu.SemaphoreType.DMA,
            pltpu.SemaphoreType.DMA,
            pltpu.SemaphoreType.DMA,
            pltpu.SemaphoreType.DMA,
            pltpu.SemaphoreType.DMA,
            pltpu.SemaphoreType.DMA,
            pltpu.SemaphoreType.DMA,
            pltpu.SemaphoreType.DMA,
            pltpu.SemaphoreType.DMA,
        ],
        compiler_params=pltpu.CompilerParams(use_tc_tiling_on_sc=False),
    )(tabA, tabB, epk, zrows, mvec)


def _s3_body(acc_ref, b1_ref, r_ref, w2_ref, s2_ref, d2_ref,
             tabA_ref, tabB_ref, mx_ref):
    i = pl.program_id(0)
    num = acc_ref[0] + acc_ref[1]
    den = num[:, D1:D1 + H1]
    dw = jnp.dot(den, r_ref[...], preferred_element_type=_f32)
    o1 = num[:, :D1] / (dw + 1e-16) + b1_ref[...]
    o1 = jnp.where(o1 > 0, o1, jnp.exp(o1) - 1.0)
    h2 = jnp.dot(o1, w2_ref[...], preferred_element_type=_f32)
    a2s = jnp.dot(h2, s2_ref[...], preferred_element_type=_f32)
    a2d = jnp.dot(h2, d2_ref[...], preferred_element_type=_f32)
    one = jnp.ones((BLK, 1), _f32)
    z6 = jnp.zeros((BLK, 6), _f32)
    z9 = jnp.zeros((BLK, 9), _f32)
    tabA_ref[...] = jnp.concatenate([h2, one, a2s, z6], axis=1)
    tabB_ref[...] = jnp.concatenate([z9, a2d, z6], axis=1)
    m = jnp.concatenate(
        [jnp.max(a2s, axis=0, keepdims=True),
         jnp.max(a2d, axis=0, keepdims=True),
         jnp.zeros((1, 14), _f32)], axis=1)

    @pl.when(i == 0)
    def _():
        mx_ref[...] = m

    @pl.when(i > 0)
    def _():
        mx_ref[...] = jnp.maximum(mx_ref[...], m)


def _stage3(acc1, b1, R, W2, s2, d2):
    return pl.pallas_call(
        _s3_body,
        grid=(GRID,),
        in_specs=[
            pl.BlockSpec((NSC, BLK, RA1), lambda i: (0, i, 0)),
            pl.BlockSpec((1, D1), lambda i: (0, 0)),
            pl.BlockSpec((H1, D1), lambda i: (0, 0)),
            pl.BlockSpec((D1, NCLS), lambda i: (0, 0)),
            pl.BlockSpec((NCLS, 1), lambda i: (0, 0)),
            pl.BlockSpec((NCLS, 1), lambda i: (0, 0)),
        ],
        out_specs=[
            pl.BlockSpec((BLK, RA2), lambda i: (i, 0)),
            pl.BlockSpec((BLK, 16), lambda i: (i, 0)),
            pl.BlockSpec((1, 16), lambda i: (0, 0)),
        ],
        out_shape=[
            jax.ShapeDtypeStruct((NPAD, RA2), _f32),
            jax.ShapeDtypeStruct((NPAD, 16), _f32),
            jax.ShapeDtypeStruct((1, 16), _f32),
        ],
    )(acc1, b1, R, W2, s2, d2)


def _s5_body(acc_ref, b2_ref, out_ref):
    num = acc_ref[0] + acc_ref[1]
    den = num[:, NCLS:NCLS + 1]
    lg = num[:, :NCLS] / (den + 1e-16) + b2_ref[...]
    m = jnp.max(lg, axis=1, keepdims=True)
    ls = lg - m
    out_ref[...] = ls - jnp.log(jnp.sum(jnp.exp(ls), axis=1, keepdims=True))


def _stage5(acc2, b2):
    return pl.pallas_call(
        _s5_body,
        grid=(GRID5,),
        in_specs=[
            pl.BlockSpec((NSC, BLK5, RA2), lambda i: (0, i, 0)),
            pl.BlockSpec((1, NCLS), lambda i: (0, 0)),
        ],
        out_specs=pl.BlockSpec((BLK5, NCLS), lambda i: (i, 0)),
        out_shape=jax.ShapeDtypeStruct((N, NCLS), _f32),
    )(acc2, b2)


def kernel(x, edge_index, W1, att_src1, att_dst1, b1, W2, att_src2, att_dst2,
           b2):
    edge_index = edge_index.astype(jnp.int32)
    epw_real = E // NW
    pad_per_w = EPW - epw_real
    ei = edge_index.reshape(2, NW, epw_real)
    padrow = (N + jnp.arange(pad_per_w, dtype=jnp.int32) % (NPAD - N))
    pad = jnp.broadcast_to(padrow, (2, NW, pad_per_w))
    epk = (jnp.concatenate([ei, pad], axis=2)
           .reshape(2, NW, NBLK, CB)
           .transpose(1, 2, 0, 3)
           .reshape(NW * NBLK, 2, CB))

    xp = jnp.concatenate([x, jnp.zeros((NPAD - N, DIN), _f32)], axis=0)

    eye = jnp.eye(H1, dtype=_f32)
    As = (att_src1[:, :, None] * eye[:, None, :]).reshape(D1, H1)
    Ad = (att_dst1[:, :, None] * eye[:, None, :]).reshape(D1, H1)
    R = jnp.repeat(eye, F1, axis=1)

    tabA1, tabB1, mx1 = _stage1(xp, W1, As, Ad)
    m1 = jnp.maximum(mx1[0, :H1] + mx1[0, H1:], 0.0)
    mvec1 = jnp.concatenate([m1, jnp.zeros((8,), _f32)])
    zrows1 = jnp.zeros((ROWS, RA1), _f32)

    acc1 = _edge_pass(RA1, H1, tabA1, tabB1, epk, zrows1, mvec1)

    tabA2, tabB2, mx2 = _stage3(acc1, b1.reshape(1, D1), R, W2,
                                att_src2.reshape(NCLS, 1),
                                att_dst2.reshape(NCLS, 1))
    m2 = jnp.maximum(mx2[0, 0] + mx2[0, 1], 0.0)
    mvec2 = jnp.zeros((16,), _f32).at[9].set(m2)
    zrows2 = jnp.zeros((ROWS, RA2), _f32)

    acc2 = _edge_pass(RA2, 1, tabA2, tabB2, epk, zrows2, mvec2)

    return _stage5(acc2, b2.reshape(1, NCLS))


# speedup vs baseline: 1.2944x; 1.0230x over previous
import functools

import jax
import jax.numpy as jnp
from jax import lax
from jax.experimental import pallas as pl
from jax.experimental.pallas import tpu as pltpu
from jax.experimental.pallas import tpu_sc as plsc

N = 10000
E = 320000
DIN = 128
H1 = 8
F1 = 16
D1 = H1 * F1
NCLS = 40

RA1 = 144
RA2 = 48

NSC = 2
NTEC = 16
NW = NSC * NTEC
NPAD = 10240
ROWS = NPAD // NTEC
CB = 80
NBR = E // NW // CB
NBLK = 128
NDUM = NBLK - NBR

BLK = 512
GRID = NPAD // BLK
BLK5 = 400
GRID5 = N // BLK5

_f32 = jnp.float32


def _bcast_lane(v, k):
    idx = jnp.full((16, 1), k, jnp.int32)
    dn = lax.GatherDimensionNumbers(
        offset_dims=(), collapsed_slice_dims=(0,), start_index_map=(0,))
    return lax.gather(v, idx, dn, (1,),
                      mode=lax.GatherScatterMode.PROMISE_IN_BOUNDS)


def _s1_body(x_ref, w_ref, as_ref, ad_ref, tabA_ref, tabB_ref, mx_ref):
    i = pl.program_id(0)
    h = jnp.dot(x_ref[...], w_ref[...], preferred_element_type=_f32)
    asrc = jnp.dot(h, as_ref[...], preferred_element_type=_f32)
    adst = jnp.dot(h, ad_ref[...], preferred_element_type=_f32)
    z = jnp.zeros((BLK, 8), _f32)
    tabA_ref[...] = jnp.concatenate([h, asrc, z], axis=1)
    tabB_ref[...] = jnp.concatenate([adst, z], axis=1)
    m = jnp.concatenate([jnp.max(asrc, axis=0, keepdims=True),
                         jnp.max(adst, axis=0, keepdims=True)], axis=1)

    @pl.when(i == 0)
    def _():
        mx_ref[...] = m

    @pl.when(i > 0)
    def _():
        mx_ref[...] = jnp.maximum(mx_ref[...], m)


def _stage1(x, W1, As, Ad):
    return pl.pallas_call(
        _s1_body,
        grid=(GRID,),
        in_specs=[
            pl.BlockSpec((BLK, DIN), lambda i: (i, 0)),
            pl.BlockSpec((DIN, D1), lambda i: (0, 0)),
            pl.BlockSpec((DIN, H1), lambda i: (0, 0)),
            pl.BlockSpec((DIN, H1), lambda i: (0, 0)),
        ],
        out_specs=[
            pl.BlockSpec((BLK, RA1), lambda i: (i, 0)),
            pl.BlockSpec((BLK, 16), lambda i: (i, 0)),
            pl.BlockSpec((1, 16), lambda i: (0, 0)),
        ],
        out_shape=[
            jax.ShapeDtypeStruct((NPAD, RA1), _f32),
            jax.ShapeDtypeStruct((NPAD, 16), _f32),
            jax.ShapeDtypeStruct((1, 16), _f32),
        ],
    )(x, W1, As, Ad)


def _edge_kernel_body(row_w, head_slots, tabA, tabB, ereal, edum, zrows, mvec,
                      out,
                      ei0, ei1, ei2, ei3, rowA0, rowA1, rowB0, rowB1, mv, acc,
                      semI0, semI1, semI2, semI3,
                      semA0, semA1, semB0, semB1, semS0, semS1):
    c = lax.axis_index("c")
    s = lax.axis_index("s")
    w = c * NTEC + s
    pltpu.sync_copy(zrows, acc.at[pl.ds(s * ROWS, ROWS)])
    pltpu.sync_copy(mvec, mv)
    plsc.subcore_barrier()
    mvv = mv[...]
    asl = row_w - 16

    ei = (ei0, ei1, ei2, ei3)
    semI = (semI0, semI1, semI2, semI3)
    rowA = (rowA0, rowA1)
    rowB = (rowB0, rowB1)
    semA = (semA0, semA1)
    semB = (semB0, semB1)
    semS = (semS0, semS1)

    def idx_start(b, j):
        @pl.when(b < NBR)
        def _():
            pltpu.async_copy(ereal.at[0, w, b], ei[j].at[0], semI[j])
            pltpu.async_copy(ereal.at[1, w, b], ei[j].at[1], semI[j])

        @pl.when(b >= NBR)
        def _():
            pltpu.async_copy(edum.at[0, b - NBR], ei[j].at[0], semI[j])
            pltpu.async_copy(edum.at[1, b - NBR], ei[j].at[1], semI[j])

    def idx_wait(j):
        pltpu.make_async_copy(edum.at[0, 0], ei[j].at[0], semI[j]).wait()
        pltpu.make_async_copy(edum.at[1, 0], ei[j].at[1], semI[j]).wait()

    def gatherA(b, p, j):
        return pltpu.make_async_copy(tabA.at[ei[j].at[0]], rowA[p], semA[p])

    def gatherB(b, p, j):
        return pltpu.make_async_copy(tabB.at[ei[j].at[1]], rowB[p], semB[p])

    def scatter(b, p, j):
        return pltpu.make_async_copy(rowA[p], acc.at[ei[j].at[1]], semS[p])

    def compute(p):
        ra_ref = rowA[p]
        rb_ref = rowB[p]

        @plsc.parallel_loop(0, CB, step=1, unroll=2)
        def edge(e):
            ra = ra_ref[e, pl.ds(asl, 16)]
            rb = rb_ref[e, pl.ds(0, 16)]
            sv = ra + rb
            al = jnp.where(sv > 0, sv, 0.2 * sv)
            p_ = jnp.exp(al - mvv)
            if head_slots == 8:
                ra_ref[e, pl.ds(asl, 16)] = p_
                for k in range(8):
                    pk = _bcast_lane(p_, k)
                    ra_ref[e, pl.ds(k * 16, 16)] = (
                        ra_ref[e, pl.ds(k * 16, 16)] * pk)
            else:
                pk = _bcast_lane(p_, 9)
                ra_ref[e, pl.ds(asl, 16)] = ra * pk
                for k in range((row_w - 16) // 16):
                    ra_ref[e, pl.ds(k * 16, 16)] = (
                        ra_ref[e, pl.ds(k * 16, 16)] * pk)

    idx_start(0, 0)
    idx_start(1, 1)
    idx_wait(0)
    gatherA(0, 0, 0).start()
    gatherB(0, 0, 0).start()

    def quad(i, carry):
        for u in range(4):
            b = 4 * i + u
            p = u % 2
            j = u
            jn = (u + 1) % 4
            jp = (u + 2) % 4
            jq = (u + 3) % 4

            @pl.when(b >= 1)
            def _(b=b, p=p, jq=jq):
                scatter(b - 1, 1 - p, jq).wait()

            @pl.when(b + 2 < NBLK)
            def _(b=b, jp=jp):
                idx_start(b + 2, jp)

            @pl.when(b + 1 < NBLK)
            def _(b=b, p=p, jn=jn):
                idx_wait(jn)
                gatherA(b + 1, 1 - p, jn).start()
                gatherB(b + 1, 1 - p, jn).start()

            gatherA(b, p, j).wait()
            gatherB(b, p, j).wait()
            compute(p)
            pltpu.async_copy(rowA[p], acc.at[ei[j].at[1]], semS[p], add=True)
        return carry

    lax.fori_loop(0, NBLK // 4, quad, 0)
    scatter(NBLK - 1, 1, 3).wait()
    plsc.subcore_barrier()
    pltpu.sync_copy(acc.at[pl.ds(s * ROWS, ROWS)],
                    out.at[c, pl.ds(s * ROWS, ROWS)])


def _edge_pass(row_w, head_slots, tabA, tabB, ereal, edum, zrows, mvec):
    mesh = plsc.VectorSubcoreMesh(core_axis_name="c", subcore_axis_name="s",
                                  num_cores=NSC, num_subcores=NTEC)
    body = functools.partial(_edge_kernel_body, row_w, head_slots)
    return pl.kernel(
        body,
        out_type=jax.ShapeDtypeStruct((NSC, NPAD, row_w), _f32),
        mesh=mesh,
        scratch_types=[
            pltpu.VMEM((2, CB), jnp.int32),
            pltpu.VMEM((2, CB), jnp.int32),
            pltpu.VMEM((2, CB), jnp.int32),
            pltpu.VMEM((2, CB), jnp.int32),
            pltpu.VMEM((CB, row_w), _f32),
            pltpu.VMEM((CB, row_w), _f32),
            pltpu.VMEM((CB, 16), _f32),
            pltpu.VMEM((CB, 16), _f32),
            pltpu.VMEM((16,), _f32),
            pltpu.VMEM_SHARED((NPAD, row_w), _f32),
            pltpu.SemaphoreType.DMA,
            pltpu.SemaphoreType.DMA,
            pltpu.SemaphoreType.DMA,
            pltpu.SemaphoreType.DMA,
            pltpu.SemaphoreType.DMA,
            pltpu.SemaphoreType.DMA,
            pltpu.SemaphoreType.DMA,
            pltpu.SemaphoreType.DMA,
            pltpu.SemaphoreType.DMA,
            pltpu.SemaphoreType.DMA,
        ],
        compiler_params=pltpu.CompilerParams(use_tc_tiling_on_sc=False),
    )(tabA, tabB, ereal, edum, zrows, mvec)


def _s3_body(acc_ref, b1_ref, r_ref, w2_ref, s2_ref, d2_ref,
             tabA_ref, tabB_ref, mx_ref):
    i = pl.program_id(0)
    num = acc_ref[...]
    den = num[:, D1:D1 + H1]
    dw = jnp.dot(den, r_ref[...], preferred_element_type=_f32)
    o1 = num[:, :D1] / (dw + 1e-16) + b1_ref[...]
    o1 = jnp.where(o1 > 0, o1, jnp.exp(o1) - 1.0)
    h2 = jnp.dot(o1, w2_ref[...], preferred_element_type=_f32)
    a2s = jnp.dot(h2, s2_ref[...], preferred_element_type=_f32)
    a2d = jnp.dot(h2, d2_ref[...], preferred_element_type=_f32)
    one = jnp.ones((BLK, 1), _f32)
    z6 = jnp.zeros((BLK, 6), _f32)
    z9 = jnp.zeros((BLK, 9), _f32)
    tabA_ref[...] = jnp.concatenate([h2, one, a2s, z6], axis=1)
    tabB_ref[...] = jnp.concatenate([z9, a2d, z6], axis=1)
    m = jnp.concatenate(
        [jnp.max(a2s, axis=0, keepdims=True),
         jnp.max(a2d, axis=0, keepdims=True),
         jnp.zeros((1, 14), _f32)], axis=1)

    @pl.when(i == 0)
    def _():
        mx_ref[...] = m

    @pl.when(i > 0)
    def _():
        mx_ref[...] = jnp.maximum(mx_ref[...], m)


def _stage3(acc1, b1, R, W2, s2, d2):
    return pl.pallas_call(
        _s3_body,
        grid=(GRID,),
        in_specs=[
            pl.BlockSpec((BLK, RA1), lambda i: (i, 0)),
            pl.BlockSpec((1, D1), lambda i: (0, 0)),
            pl.BlockSpec((H1, D1), lambda i: (0, 0)),
            pl.BlockSpec((D1, NCLS), lambda i: (0, 0)),
            pl.BlockSpec((NCLS, 1), lambda i: (0, 0)),
            pl.BlockSpec((NCLS, 1), lambda i: (0, 0)),
        ],
        out_specs=[
            pl.BlockSpec((BLK, RA2), lambda i: (i, 0)),
            pl.BlockSpec((BLK, 16), lambda i: (i, 0)),
            pl.BlockSpec((1, 16), lambda i: (0, 0)),
        ],
        out_shape=[
            jax.ShapeDtypeStruct((NPAD, RA2), _f32),
            jax.ShapeDtypeStruct((NPAD, 16), _f32),
            jax.ShapeDtypeStruct((1, 16), _f32),
        ],
    )(acc1, b1, R, W2, s2, d2)


def _s5_body(acc_ref, b2_ref, out_ref):
    num = acc_ref[...]
    den = num[:, NCLS:NCLS + 1]
    lg = num[:, :NCLS] / (den + 1e-16) + b2_ref[...]
    m = jnp.max(lg, axis=1, keepdims=True)
    ls = lg - m
    out_ref[...] = ls - jnp.log(jnp.sum(jnp.exp(ls), axis=1, keepdims=True))


def _stage5(acc2, b2):
    return pl.pallas_call(
        _s5_body,
        grid=(GRID5,),
        in_specs=[
            pl.BlockSpec((BLK5, RA2), lambda i: (i, 0)),
            pl.BlockSpec((1, NCLS), lambda i: (0, 0)),
        ],
        out_specs=pl.BlockSpec((BLK5, NCLS), lambda i: (i, 0)),
        out_shape=jax.ShapeDtypeStruct((N, NCLS), _f32),
    )(acc2, b2)


def kernel(x, edge_index, W1, att_src1, att_dst1, b1, W2, att_src2, att_dst2,
           b2):
    edge_index = edge_index.astype(jnp.int32)
    ereal = edge_index.reshape(2, NW, NBR, CB)
    padrow = N + jnp.arange(NDUM * CB, dtype=jnp.int32) % (NPAD - N)
    edum = jnp.broadcast_to(padrow.reshape(NDUM, CB), (2, NDUM, CB))

    xp = jnp.concatenate([x, jnp.zeros((NPAD - N, DIN), _f32)], axis=0)

    eye = jnp.eye(H1, dtype=_f32)
    As = (att_src1[:, :, None] * eye[:, None, :]).reshape(D1, H1)
    Ad = (att_dst1[:, :, None] * eye[:, None, :]).reshape(D1, H1)
    R = jnp.repeat(eye, F1, axis=1)

    tabA1, tabB1, mx1 = _stage1(xp, W1, As, Ad)
    m1 = jnp.maximum(mx1[0, :H1] + mx1[0, H1:], 0.0)
    mvec1 = jnp.concatenate([m1, jnp.zeros((8,), _f32)])
    zrows1 = jnp.zeros((ROWS, RA1), _f32)

    acc1 = _edge_pass(RA1, H1, tabA1, tabB1, ereal, edum, zrows1, mvec1)

    tabA2, tabB2, mx2 = _stage3(acc1[0] + acc1[1], b1.reshape(1, D1), R, W2,
                                att_src2.reshape(NCLS, 1),
                                att_dst2.reshape(NCLS, 1))
    m2 = jnp.maximum(mx2[0, 0] + mx2[0, 1], 0.0)
    mvec2 = jnp.zeros((16,), _f32).at[9].set(m2)
    zrows2 = jnp.zeros((ROWS, RA2), _f32)

    acc2 = _edge_pass(RA2, 1, tabA2, tabB2, ereal, edum, zrows2, mvec2)

    return _stage5(acc2[0] + acc2[1], b2.reshape(1, NCLS))


# speedup vs baseline: 1.4442x; 1.1158x over previous
import functools

import jax
import jax.numpy as jnp
from jax import lax
from jax.experimental import pallas as pl
from jax.experimental.pallas import tpu as pltpu
from jax.experimental.pallas import tpu_sc as plsc

N = 10000
E = 320000
DIN = 128
H1 = 8
F1 = 16
D1 = H1 * F1
NCLS = 40

RA1 = 144
RA2 = 48

NSC = 2
NTEC = 16
NW = NSC * NTEC
NPAD = 10240
ROWS = NPAD // NTEC
CB = 80
NBR = E // NW // CB
NBLK = 128
NDUM = NBLK - NBR

BLK = 2048
GRID = NPAD // BLK
BLK5 = 2000
GRID5 = N // BLK5

_f32 = jnp.float32


def _bcast_lane(v, k):
    idx = jnp.full((16, 1), k, jnp.int32)
    dn = lax.GatherDimensionNumbers(
        offset_dims=(), collapsed_slice_dims=(0,), start_index_map=(0,))
    return lax.gather(v, idx, dn, (1,),
                      mode=lax.GatherScatterMode.PROMISE_IN_BOUNDS)


def _s1_body(x_ref, w_ref, as_ref, ad_ref, tabA_ref, tabB_ref, mx_ref):
    i = pl.program_id(0)
    h = jnp.dot(x_ref[...], w_ref[...], preferred_element_type=_f32)
    row = i * BLK + lax.broadcasted_iota(jnp.int32, (BLK, 1), 0)
    h = jnp.where(row < N, h, 0.0)
    asrc = jnp.dot(h, as_ref[...], preferred_element_type=_f32)
    adst = jnp.dot(h, ad_ref[...], preferred_element_type=_f32)
    z = jnp.zeros((BLK, 8), _f32)
    tabA_ref[...] = jnp.concatenate([h, asrc, z], axis=1)
    tabB_ref[...] = jnp.concatenate([adst, z], axis=1)
    m = jnp.concatenate([jnp.max(asrc, axis=0, keepdims=True),
                         jnp.max(adst, axis=0, keepdims=True)], axis=1)

    @pl.when(i == 0)
    def _():
        mx_ref[...] = m

    @pl.when(i > 0)
    def _():
        mx_ref[...] = jnp.maximum(mx_ref[...], m)


def _stage1(x, W1, As, Ad):
    return pl.pallas_call(
        _s1_body,
        grid=(GRID,),
        in_specs=[
            pl.BlockSpec((BLK, DIN), lambda i: (i, 0)),
            pl.BlockSpec((DIN, D1), lambda i: (0, 0)),
            pl.BlockSpec((DIN, H1), lambda i: (0, 0)),
            pl.BlockSpec((DIN, H1), lambda i: (0, 0)),
        ],
        out_specs=[
            pl.BlockSpec((BLK, RA1), lambda i: (i, 0)),
            pl.BlockSpec((BLK, 16), lambda i: (i, 0)),
            pl.BlockSpec((1, 16), lambda i: (0, 0)),
        ],
        out_shape=[
            jax.ShapeDtypeStruct((NPAD, RA1), _f32),
            jax.ShapeDtypeStruct((NPAD, 16), _f32),
            jax.ShapeDtypeStruct((1, 16), _f32),
        ],
    )(x, W1, As, Ad)


def _edge_kernel_body(row_w, head_slots, tabA, tabB, ereal, edum, zrows, mvec,
                      out,
                      ei0, ei1, ei2, ei3, rowA0, rowA1, rowB0, rowB1, mv, acc,
                      semI0, semI1, semI2, semI3,
                      semA0, semA1, semB0, semB1, semS0, semS1):
    c = lax.axis_index("c")
    s = lax.axis_index("s")
    w = c * NTEC + s
    pltpu.sync_copy(zrows, acc.at[pl.ds(s * ROWS, ROWS)])
    pltpu.sync_copy(mvec, mv)
    plsc.subcore_barrier()
    mvv = mv[...]
    asl = row_w - 16

    ei = (ei0, ei1, ei2, ei3)
    semI = (semI0, semI1, semI2, semI3)
    rowA = (rowA0, rowA1)
    rowB = (rowB0, rowB1)
    semA = (semA0, semA1)
    semB = (semB0, semB1)
    semS = (semS0, semS1)

    def idx_start(b, j):
        @pl.when(b < NBR)
        def _():
            pltpu.async_copy(ereal.at[0, w, b], ei[j].at[0], semI[j])
            pltpu.async_copy(ereal.at[1, w, b], ei[j].at[1], semI[j])

        @pl.when(b >= NBR)
        def _():
            pltpu.async_copy(edum.at[0, b - NBR], ei[j].at[0], semI[j])
            pltpu.async_copy(edum.at[1, b - NBR], ei[j].at[1], semI[j])

    def idx_wait(j):
        pltpu.make_async_copy(edum.at[0, 0], ei[j].at[0], semI[j]).wait()
        pltpu.make_async_copy(edum.at[1, 0], ei[j].at[1], semI[j]).wait()

    def gatherA(b, p, j):
        return pltpu.make_async_copy(tabA.at[ei[j].at[0]], rowA[p], semA[p])

    def gatherB(b, p, j):
        return pltpu.make_async_copy(tabB.at[ei[j].at[1]], rowB[p], semB[p])

    def scatter(b, p, j):
        return pltpu.make_async_copy(rowA[p], acc.at[ei[j].at[1]], semS[p])

    def compute(p):
        ra_ref = rowA[p]
        rb_ref = rowB[p]

        @plsc.parallel_loop(0, CB, step=1, unroll=2)
        def edge(e):
            ra = ra_ref[e, pl.ds(asl, 16)]
            rb = rb_ref[e, pl.ds(0, 16)]
            sv = ra + rb
            al = jnp.where(sv > 0, sv, 0.2 * sv)
            p_ = jnp.exp(al - mvv)
            if head_slots == 8:
                ra_ref[e, pl.ds(asl, 16)] = p_
                for k in range(8):
                    pk = _bcast_lane(p_, k)
                    ra_ref[e, pl.ds(k * 16, 16)] = (
                        ra_ref[e, pl.ds(k * 16, 16)] * pk)
            else:
                pk = _bcast_lane(p_, 9)
                ra_ref[e, pl.ds(asl, 16)] = ra * pk
                for k in range((row_w - 16) // 16):
                    ra_ref[e, pl.ds(k * 16, 16)] = (
                        ra_ref[e, pl.ds(k * 16, 16)] * pk)

    idx_start(0, 0)
    idx_start(1, 1)
    idx_wait(0)
    gatherA(0, 0, 0).start()
    gatherB(0, 0, 0).start()

    def quad(i, carry):
        for u in range(4):
            b = 4 * i + u
            p = u % 2
            j = u
            jn = (u + 1) % 4
            jp = (u + 2) % 4
            jq = (u + 3) % 4

            @pl.when(b >= 1)
            def _(b=b, p=p, jq=jq):
                scatter(b - 1, 1 - p, jq).wait()

            @pl.when(b + 2 < NBLK)
            def _(b=b, jp=jp):
                idx_start(b + 2, jp)

            @pl.when(b + 1 < NBLK)
            def _(b=b, p=p, jn=jn):
                idx_wait(jn)
                gatherA(b + 1, 1 - p, jn).start()
                gatherB(b + 1, 1 - p, jn).start()

            gatherA(b, p, j).wait()
            gatherB(b, p, j).wait()
            compute(p)
            pltpu.async_copy(rowA[p], acc.at[ei[j].at[1]], semS[p], add=True)
        return carry

    lax.fori_loop(0, NBLK // 4, quad, 0)
    scatter(NBLK - 1, 1, 3).wait()
    plsc.subcore_barrier()
    pltpu.sync_copy(acc.at[pl.ds(s * ROWS, ROWS)],
                    out.at[c, pl.ds(s * ROWS, ROWS)])


def _edge_pass(row_w, head_slots, tabA, tabB, ereal, edum, zrows, mvec):
    mesh = plsc.VectorSubcoreMesh(core_axis_name="c", subcore_axis_name="s",
                                  num_cores=NSC, num_subcores=NTEC)
    body = functools.partial(_edge_kernel_body, row_w, head_slots)
    return pl.kernel(
        body,
        out_type=jax.ShapeDtypeStruct((NSC, NPAD, row_w), _f32),
        mesh=mesh,
        scratch_types=[
            pltpu.VMEM((2, CB), jnp.int32),
            pltpu.VMEM((2, CB), jnp.int32),
            pltpu.VMEM((2, CB), jnp.int32),
            pltpu.VMEM((2, CB), jnp.int32),
            pltpu.VMEM((CB, row_w), _f32),
            pltpu.VMEM((CB, row_w), _f32),
            pltpu.VMEM((CB, 16), _f32),
            pltpu.VMEM((CB, 16), _f32),
            pltpu.VMEM((16,), _f32),
            pltpu.VMEM_SHARED((NPAD, row_w), _f32),
            pltpu.SemaphoreType.DMA,
            pltpu.SemaphoreType.DMA,
            pltpu.SemaphoreType.DMA,
            pltpu.SemaphoreType.DMA,
            pltpu.SemaphoreType.DMA,
            pltpu.SemaphoreType.DMA,
            pltpu.SemaphoreType.DMA,
            pltpu.SemaphoreType.DMA,
            pltpu.SemaphoreType.DMA,
            pltpu.SemaphoreType.DMA,
        ],
        compiler_params=pltpu.CompilerParams(use_tc_tiling_on_sc=False),
    )(tabA, tabB, ereal, edum, zrows, mvec)


def _s3_body(acc_ref, b1_ref, r_ref, w2_ref, s2_ref, d2_ref,
             tabA_ref, tabB_ref, mx_ref):
    i = pl.program_id(0)
    num = acc_ref[0] + acc_ref[1]
    den = num[:, D1:D1 + H1]
    dw = jnp.dot(den, r_ref[...], preferred_element_type=_f32)
    o1 = num[:, :D1] / (dw + 1e-16) + b1_ref[...]
    o1 = jnp.where(o1 > 0, o1, jnp.exp(o1) - 1.0)
    h2 = jnp.dot(o1, w2_ref[...], preferred_element_type=_f32)
    a2s = jnp.dot(h2, s2_ref[...], preferred_element_type=_f32)
    a2d = jnp.dot(h2, d2_ref[...], preferred_element_type=_f32)
    one = jnp.ones((BLK, 1), _f32)
    z6 = jnp.zeros((BLK, 6), _f32)
    z9 = jnp.zeros((BLK, 9), _f32)
    tabA_ref[...] = jnp.concatenate([h2, one, a2s, z6], axis=1)
    tabB_ref[...] = jnp.concatenate([z9, a2d, z6], axis=1)
    m = jnp.concatenate(
        [jnp.max(a2s, axis=0, keepdims=True),
         jnp.max(a2d, axis=0, keepdims=True),
         jnp.zeros((1, 14), _f32)], axis=1)

    @pl.when(i == 0)
    def _():
        mx_ref[...] = m

    @pl.when(i > 0)
    def _():
        mx_ref[...] = jnp.maximum(mx_ref[...], m)


def _stage3(acc1, b1, R, W2, s2, d2):
    return pl.pallas_call(
        _s3_body,
        grid=(GRID,),
        in_specs=[
            pl.BlockSpec((NSC, BLK, RA1), lambda i: (0, i, 0)),
            pl.BlockSpec((1, D1), lambda i: (0, 0)),
            pl.BlockSpec((H1, D1), lambda i: (0, 0)),
            pl.BlockSpec((D1, NCLS), lambda i: (0, 0)),
            pl.BlockSpec((NCLS, 1), lambda i: (0, 0)),
            pl.BlockSpec((NCLS, 1), lambda i: (0, 0)),
        ],
        out_specs=[
            pl.BlockSpec((BLK, RA2), lambda i: (i, 0)),
            pl.BlockSpec((BLK, 16), lambda i: (i, 0)),
            pl.BlockSpec((1, 16), lambda i: (0, 0)),
        ],
        out_shape=[
            jax.ShapeDtypeStruct((NPAD, RA2), _f32),
            jax.ShapeDtypeStruct((NPAD, 16), _f32),
            jax.ShapeDtypeStruct((1, 16), _f32),
        ],
    )(acc1, b1, R, W2, s2, d2)


def _s5_body(acc_ref, b2_ref, out_ref):
    num = acc_ref[0] + acc_ref[1]
    den = num[:, NCLS:NCLS + 1]
    lg = num[:, :NCLS] / (den + 1e-16) + b2_ref[...]
    m = jnp.max(lg, axis=1, keepdims=True)
    ls = lg - m
    out_ref[...] = ls - jnp.log(jnp.sum(jnp.exp(ls), axis=1, keepdims=True))


def _stage5(acc2, b2):
    return pl.pallas_call(
        _s5_body,
        grid=(GRID5,),
        in_specs=[
            pl.BlockSpec((NSC, BLK5, RA2), lambda i: (0, i, 0)),
            pl.BlockSpec((1, NCLS), lambda i: (0, 0)),
        ],
        out_specs=pl.BlockSpec((BLK5, NCLS), lambda i: (i, 0)),
        out_shape=jax.ShapeDtypeStruct((N, NCLS), _f32),
    )(acc2, b2)


def kernel(x, edge_index, W1, att_src1, att_dst1, b1, W2, att_src2, att_dst2,
           b2):
    edge_index = edge_index.astype(jnp.int32)
    ereal = edge_index.reshape(2, NW, NBR, CB)
    padrow = N + jnp.arange(NDUM * CB, dtype=jnp.int32) % (NPAD - N)
    edum = jnp.broadcast_to(padrow.reshape(NDUM, CB), (2, NDUM, CB))

    eye = jnp.eye(H1, dtype=_f32)
    As = (att_src1[:, :, None] * eye[:, None, :]).reshape(D1, H1)
    Ad = (att_dst1[:, :, None] * eye[:, None, :]).reshape(D1, H1)
    R = jnp.repeat(eye, F1, axis=1)

    tabA1, tabB1, mx1 = _stage1(x, W1, As, Ad)
    m1 = jnp.maximum(mx1[0, :H1] + mx1[0, H1:], 0.0)
    mvec1 = jnp.concatenate([m1, jnp.zeros((8,), _f32)])
    zrows1 = jnp.zeros((ROWS, RA1), _f32)

    acc1 = _edge_pass(RA1, H1, tabA1, tabB1, ereal, edum, zrows1, mvec1)

    tabA2, tabB2, mx2 = _stage3(acc1, b1.reshape(1, D1), R, W2,
                                att_src2.reshape(NCLS, 1),
                                att_dst2.reshape(NCLS, 1))
    m2 = jnp.maximum(mx2[0, 0] + mx2[0, 1], 0.0)
    mvec2 = jnp.zeros((16,), _f32).at[9].set(m2)
    zrows2 = jnp.zeros((ROWS, RA2), _f32)

    acc2 = _edge_pass(RA2, 1, tabA2, tabB2, ereal, edum, zrows2, mvec2)

    return _stage5(acc2, b2.reshape(1, NCLS))


# speedup vs baseline: 1.4454x; 1.0008x over previous
import functools

import jax
import jax.numpy as jnp
from jax import lax
from jax.experimental import pallas as pl
from jax.experimental.pallas import tpu as pltpu
from jax.experimental.pallas import tpu_sc as plsc

N = 10000
E = 320000
DIN = 128
H1 = 8
F1 = 16
D1 = H1 * F1
NCLS = 40

RA1 = 144
RA2 = 48

NSC = 2
NTEC = 16
NW = NSC * NTEC
NPAD = 10240
ROWS = NPAD // NTEC
CB = 80
NBR = E // NW // CB
NBLK = 128
NDUM = NBLK - NBR

BLK = 2048
GRID = NPAD // BLK
BLK5 = 2000
GRID5 = N // BLK5

_f32 = jnp.float32


def _bcast_lane(v, k):
    idx = jnp.full((16, 1), k, jnp.int32)
    dn = lax.GatherDimensionNumbers(
        offset_dims=(), collapsed_slice_dims=(0,), start_index_map=(0,))
    return lax.gather(v, idx, dn, (1,),
                      mode=lax.GatherScatterMode.PROMISE_IN_BOUNDS)


def _s1_body(x_ref, w_ref, as_ref, ad_ref, tabA_ref, tabB_ref, mx_ref):
    i = pl.program_id(0)
    h = jnp.dot(x_ref[...], w_ref[...], preferred_element_type=_f32)
    row = i * BLK + lax.broadcasted_iota(jnp.int32, (BLK, 1), 0)
    h = jnp.where(row < N, h, 0.0)
    asrc = jnp.dot(h, as_ref[...], preferred_element_type=_f32)
    adst = jnp.dot(h, ad_ref[...], preferred_element_type=_f32)
    z = jnp.zeros((BLK, 8), _f32)
    tabA_ref[...] = jnp.concatenate([h, asrc, z], axis=1)
    tabB_ref[...] = jnp.concatenate([adst, z], axis=1)
    m = jnp.concatenate([jnp.max(asrc, axis=0, keepdims=True),
                         jnp.max(adst, axis=0, keepdims=True)], axis=1)

    @pl.when(i == 0)
    def _():
        mx_ref[...] = m

    @pl.when(i > 0)
    def _():
        mx_ref[...] = jnp.maximum(mx_ref[...], m)


def _stage1(x, W1, As, Ad):
    return pl.pallas_call(
        _s1_body,
        grid=(GRID,),
        in_specs=[
            pl.BlockSpec((BLK, DIN), lambda i: (i, 0)),
            pl.BlockSpec((DIN, D1), lambda i: (0, 0)),
            pl.BlockSpec((DIN, H1), lambda i: (0, 0)),
            pl.BlockSpec((DIN, H1), lambda i: (0, 0)),
        ],
        out_specs=[
            pl.BlockSpec((BLK, RA1), lambda i: (i, 0)),
            pl.BlockSpec((BLK, 16), lambda i: (i, 0)),
            pl.BlockSpec((1, 16), lambda i: (0, 0)),
        ],
        out_shape=[
            jax.ShapeDtypeStruct((NPAD, RA1), _f32),
            jax.ShapeDtypeStruct((NPAD, 16), _f32),
            jax.ShapeDtypeStruct((1, 16), _f32),
        ],
    )(x, W1, As, Ad)


def _edge_kernel_body(row_w, head_slots, tabA, tabB, ereal, edum, zrows, mvec,
                      out,
                      ei0, ei1, ei2, ei3, rowA0, rowA1, rowB0, rowB1, mv, acc,
                      semI0, semI1, semI2, semI3,
                      semA0, semA1, semB0, semB1, semS0, semS1):
    c = lax.axis_index("c")
    s = lax.axis_index("s")
    w = c * NTEC + s
    pltpu.sync_copy(zrows, acc.at[pl.ds(s * ROWS, ROWS)])
    pltpu.sync_copy(mvec, mv)
    plsc.subcore_barrier()
    mvv = mv[...]
    asl = row_w - 16

    ei = (ei0, ei1, ei2, ei3)
    semI = (semI0, semI1, semI2, semI3)
    rowA = (rowA0, rowA1)
    rowB = (rowB0, rowB1)
    semA = (semA0, semA1)
    semB = (semB0, semB1)
    semS = (semS0, semS1)

    def idx_start(b, j):
        @pl.when(b < NBR)
        def _():
            pltpu.async_copy(ereal.at[0, w, b], ei[j].at[0], semI[j])
            pltpu.async_copy(ereal.at[1, w, b], ei[j].at[1], semI[j])

        @pl.when(b >= NBR)
        def _():
            pltpu.async_copy(edum.at[0, b - NBR], ei[j].at[0], semI[j])
            pltpu.async_copy(edum.at[1, b - NBR], ei[j].at[1], semI[j])

    def idx_wait(j):
        pltpu.make_async_copy(edum.at[0, 0], ei[j].at[0], semI[j]).wait()
        pltpu.make_async_copy(edum.at[1, 0], ei[j].at[1], semI[j]).wait()

    def gatherA(b, p, j):
        return pltpu.make_async_copy(tabA.at[ei[j].at[0]], rowA[p], semA[p])

    def gatherB(b, p, j):
        return pltpu.make_async_copy(tabB.at[ei[j].at[1]], rowB[p], semB[p])

    def scatter(b, p, j):
        return pltpu.make_async_copy(rowA[p], acc.at[ei[j].at[1]], semS[p])

    def compute(p):
        ra_ref = rowA[p]
        rb_ref = rowB[p]

        @plsc.parallel_loop(0, CB, step=1, unroll=4 if head_slots == 1 else 2)
        def edge(e):
            ra = ra_ref[e, pl.ds(asl, 16)]
            rb = rb_ref[e, pl.ds(0, 16)]
            sv = ra + rb
            al = jnp.where(sv > 0, sv, 0.2 * sv)
            p_ = jnp.exp(al - mvv)
            if head_slots == 8:
                ra_ref[e, pl.ds(asl, 16)] = p_
                for k in range(8):
                    pk = _bcast_lane(p_, k)
                    ra_ref[e, pl.ds(k * 16, 16)] = (
                        ra_ref[e, pl.ds(k * 16, 16)] * pk)
            else:
                pk = _bcast_lane(p_, 9)
                ra_ref[e, pl.ds(asl, 16)] = ra * pk
                for k in range((row_w - 16) // 16):
                    ra_ref[e, pl.ds(k * 16, 16)] = (
                        ra_ref[e, pl.ds(k * 16, 16)] * pk)

    idx_start(0, 0)
    idx_start(1, 1)
    idx_wait(0)
    gatherA(0, 0, 0).start()
    gatherB(0, 0, 0).start()

    def quad(i, carry):
        for u in range(4):
            b = 4 * i + u
            p = u % 2
            j = u
            jn = (u + 1) % 4
            jp = (u + 2) % 4
            jq = (u + 3) % 4

            @pl.when(b >= 1)
            def _(b=b, p=p, jq=jq):
                scatter(b - 1, 1 - p, jq).wait()

            @pl.when(b + 2 < NBLK)
            def _(b=b, jp=jp):
                idx_start(b + 2, jp)

            @pl.when(b + 1 < NBLK)
            def _(b=b, p=p, jn=jn):
                idx_wait(jn)
                gatherA(b + 1, 1 - p, jn).start()
                gatherB(b + 1, 1 - p, jn).start()

            gatherA(b, p, j).wait()
            gatherB(b, p, j).wait()
            compute(p)
            pltpu.async_copy(rowA[p], acc.at[ei[j].at[1]], semS[p], add=True)
        return carry

    lax.fori_loop(0, NBLK // 4, quad, 0)
    scatter(NBLK - 1, 1, 3).wait()
    plsc.subcore_barrier()
    pltpu.sync_copy(acc.at[pl.ds(s * ROWS, ROWS)],
                    out.at[c, pl.ds(s * ROWS, ROWS)])


def _edge_pass(row_w, head_slots, tabA, tabB, ereal, edum, zrows, mvec):
    mesh = plsc.VectorSubcoreMesh(core_axis_name="c", subcore_axis_name="s",
                                  num_cores=NSC, num_subcores=NTEC)
    body = functools.partial(_edge_kernel_body, row_w, head_slots)
    return pl.kernel(
        body,
        out_type=jax.ShapeDtypeStruct((NSC, NPAD, row_w), _f32),
        mesh=mesh,
        scratch_types=[
            pltpu.VMEM((2, CB), jnp.int32),
            pltpu.VMEM((2, CB), jnp.int32),
            pltpu.VMEM((2, CB), jnp.int32),
            pltpu.VMEM((2, CB), jnp.int32),
            pltpu.VMEM((CB, row_w), _f32),
            pltpu.VMEM((CB, row_w), _f32),
            pltpu.VMEM((CB, 16), _f32),
            pltpu.VMEM((CB, 16), _f32),
            pltpu.VMEM((16,), _f32),
            pltpu.VMEM_SHARED((NPAD, row_w), _f32),
            pltpu.SemaphoreType.DMA,
            pltpu.SemaphoreType.DMA,
            pltpu.SemaphoreType.DMA,
            pltpu.SemaphoreType.DMA,
            pltpu.SemaphoreType.DMA,
            pltpu.SemaphoreType.DMA,
            pltpu.SemaphoreType.DMA,
            pltpu.SemaphoreType.DMA,
            pltpu.SemaphoreType.DMA,
            pltpu.SemaphoreType.DMA,
        ],
        compiler_params=pltpu.CompilerParams(use_tc_tiling_on_sc=False),
    )(tabA, tabB, ereal, edum, zrows, mvec)


def _s3_body(acc_ref, b1_ref, r_ref, w2_ref, s2_ref, d2_ref,
             tabA_ref, tabB_ref, mx_ref):
    i = pl.program_id(0)
    num = acc_ref[0] + acc_ref[1]
    den = num[:, D1:D1 + H1]
    dw = jnp.dot(den, r_ref[...], preferred_element_type=_f32)
    o1 = num[:, :D1] / (dw + 1e-16) + b1_ref[...]
    o1 = jnp.where(o1 > 0, o1, jnp.exp(o1) - 1.0)
    h2 = jnp.dot(o1, w2_ref[...], preferred_element_type=_f32)
    a2s = jnp.dot(h2, s2_ref[...], preferred_element_type=_f32)
    a2d = jnp.dot(h2, d2_ref[...], preferred_element_type=_f32)
    one = jnp.ones((BLK, 1), _f32)
    z6 = jnp.zeros((BLK, 6), _f32)
    z9 = jnp.zeros((BLK, 9), _f32)
    tabA_ref[...] = jnp.concatenate([h2, one, a2s, z6], axis=1)
    tabB_ref[...] = jnp.concatenate([z9, a2d, z6], axis=1)
    m = jnp.concatenate(
        [jnp.max(a2s, axis=0, keepdims=True),
         jnp.max(a2d, axis=0, keepdims=True),
         jnp.zeros((1, 14), _f32)], axis=1)

    @pl.when(i == 0)
    def _():
        mx_ref[...] = m

    @pl.when(i > 0)
    def _():
        mx_ref[...] = jnp.maximum(mx_ref[...], m)


def _stage3(acc1, b1, R, W2, s2, d2):
    return pl.pallas_call(
        _s3_body,
        grid=(GRID,),
        in_specs=[
            pl.BlockSpec((NSC, BLK, RA1), lambda i: (0, i, 0)),
            pl.BlockSpec((1, D1), lambda i: (0, 0)),
            pl.BlockSpec((H1, D1), lambda i: (0, 0)),
            pl.BlockSpec((D1, NCLS), lambda i: (0, 0)),
            pl.BlockSpec((NCLS, 1), lambda i: (0, 0)),
            pl.BlockSpec((NCLS, 1), lambda i: (0, 0)),
        ],
        out_specs=[
            pl.BlockSpec((BLK, RA2), lambda i: (i, 0)),
            pl.BlockSpec((BLK, 16), lambda i: (i, 0)),
            pl.BlockSpec((1, 16), lambda i: (0, 0)),
        ],
        out_shape=[
            jax.ShapeDtypeStruct((NPAD, RA2), _f32),
            jax.ShapeDtypeStruct((NPAD, 16), _f32),
            jax.ShapeDtypeStruct((1, 16), _f32),
        ],
    )(acc1, b1, R, W2, s2, d2)


def _s5_body(acc_ref, b2_ref, out_ref):
    num = acc_ref[0] + acc_ref[1]
    den = num[:, NCLS:NCLS + 1]
    lg = num[:, :NCLS] / (den + 1e-16) + b2_ref[...]
    m = jnp.max(lg, axis=1, keepdims=True)
    ls = lg - m
    out_ref[...] = ls - jnp.log(jnp.sum(jnp.exp(ls), axis=1, keepdims=True))


def _stage5(acc2, b2):
    return pl.pallas_call(
        _s5_body,
        grid=(GRID5,),
        in_specs=[
            pl.BlockSpec((NSC, BLK5, RA2), lambda i: (0, i, 0)),
            pl.BlockSpec((1, NCLS), lambda i: (0, 0)),
        ],
        out_specs=pl.BlockSpec((BLK5, NCLS), lambda i: (i, 0)),
        out_shape=jax.ShapeDtypeStruct((N, NCLS), _f32),
    )(acc2, b2)


def kernel(x, edge_index, W1, att_src1, att_dst1, b1, W2, att_src2, att_dst2,
           b2):
    edge_index = edge_index.astype(jnp.int32)
    ereal = edge_index.reshape(2, NW, NBR, CB)
    padrow = N + jnp.arange(NDUM * CB, dtype=jnp.int32) % (NPAD - N)
    edum = jnp.broadcast_to(padrow.reshape(NDUM, CB), (2, NDUM, CB))

    eye = jnp.eye(H1, dtype=_f32)
    As = (att_src1[:, :, None] * eye[:, None, :]).reshape(D1, H1)
    Ad = (att_dst1[:, :, None] * eye[:, None, :]).reshape(D1, H1)
    R = jnp.repeat(eye, F1, axis=1)

    tabA1, tabB1, mx1 = _stage1(x, W1, As, Ad)
    m1 = jnp.maximum(mx1[0, :H1] + mx1[0, H1:], 0.0)
    mvec1 = jnp.concatenate([m1, jnp.zeros((8,), _f32)])
    zrows1 = jnp.zeros((ROWS, RA1), _f32)

    acc1 = _edge_pass(RA1, H1, tabA1, tabB1, ereal, edum, zrows1, mvec1)

    tabA2, tabB2, mx2 = _stage3(acc1, b1.reshape(1, D1), R, W2,
                                att_src2.reshape(NCLS, 1),
                                att_dst2.reshape(NCLS, 1))
    m2 = jnp.maximum(mx2[0, 0] + mx2[0, 1], 0.0)
    mvec2 = jnp.zeros((16,), _f32).at[9].set(m2)
    zrows2 = jnp.zeros((ROWS, RA2), _f32)

    acc2 = _edge_pass(RA2, 1, tabA2, tabB2, ereal, edum, zrows2, mvec2)

    return _stage5(acc2, b2.reshape(1, NCLS))


# speedup vs baseline: 1.4497x; 1.0029x over previous
import functools

import jax
import jax.numpy as jnp
from jax import lax
from jax.experimental import pallas as pl
from jax.experimental.pallas import tpu as pltpu
from jax.experimental.pallas import tpu_sc as plsc

N = 10000
E = 320000
DIN = 128
H1 = 8
F1 = 16
D1 = H1 * F1
NCLS = 40

RA1 = 144
RA2 = 48

NSC = 2
NTEC = 16
NW = NSC * NTEC
NPAD = 10240
ROWS = NPAD // NTEC
CB = 80
NBR = E // NW // CB
NBLK = 128
NDUM = NBLK - NBR

BLK = 2048
GRID = NPAD // BLK
BLK5 = 2000
GRID5 = N // BLK5

_f32 = jnp.float32


def _bcast_lane(v, k):
    idx = jnp.full((16, 1), k, jnp.int32)
    dn = lax.GatherDimensionNumbers(
        offset_dims=(), collapsed_slice_dims=(0,), start_index_map=(0,))
    return lax.gather(v, idx, dn, (1,),
                      mode=lax.GatherScatterMode.PROMISE_IN_BOUNDS)


def _s1_body(x_ref, w_ref, as_ref, ad_ref, tabA_ref, tabB_ref, mx_ref):
    i = pl.program_id(0)
    h = jnp.dot(x_ref[...], w_ref[...], preferred_element_type=_f32)
    row = i * BLK + lax.broadcasted_iota(jnp.int32, (BLK, 1), 0)
    h = jnp.where(row < N, h, 0.0)
    asrc = jnp.dot(h, as_ref[...], preferred_element_type=_f32)
    adst = jnp.dot(h, ad_ref[...], preferred_element_type=_f32)
    z = jnp.zeros((BLK, 8), _f32)
    tabA_ref[...] = jnp.concatenate([h, asrc, z], axis=1)
    tabB_ref[...] = jnp.concatenate([adst, z], axis=1)
    m = jnp.concatenate([jnp.max(asrc, axis=0, keepdims=True),
                         jnp.max(adst, axis=0, keepdims=True)], axis=1)

    @pl.when(i == 0)
    def _():
        mx_ref[...] = m

    @pl.when(i > 0)
    def _():
        mx_ref[...] = jnp.maximum(mx_ref[...], m)


def _stage1(x, W1, As, Ad):
    return pl.pallas_call(
        _s1_body,
        grid=(GRID,),
        in_specs=[
            pl.BlockSpec((BLK, DIN), lambda i: (i, 0)),
            pl.BlockSpec((DIN, D1), lambda i: (0, 0)),
            pl.BlockSpec((DIN, H1), lambda i: (0, 0)),
            pl.BlockSpec((DIN, H1), lambda i: (0, 0)),
        ],
        out_specs=[
            pl.BlockSpec((BLK, RA1), lambda i: (i, 0)),
            pl.BlockSpec((BLK, 16), lambda i: (i, 0)),
            pl.BlockSpec((1, 16), lambda i: (0, 0)),
        ],
        out_shape=[
            jax.ShapeDtypeStruct((NPAD, RA1), _f32),
            jax.ShapeDtypeStruct((NPAD, 16), _f32),
            jax.ShapeDtypeStruct((1, 16), _f32),
        ],
    )(x, W1, As, Ad)


def _edge_kernel_body(row_w, head_slots, tabA, tabB, ereal, edum, zrows, mvec,
                      out,
                      ei0, ei1, ei2, ei3, rowA0, rowA1, rowB0, rowB1, mv, acc,
                      semI0, semI1, semI2, semI3,
                      semA0, semA1, semB0, semB1, semS0, semS1):
    c = lax.axis_index("c")
    s = lax.axis_index("s")
    w = c * NTEC + s
    pltpu.sync_copy(zrows, acc.at[pl.ds(s * ROWS, ROWS)])
    pltpu.sync_copy(mvec, mv)
    plsc.subcore_barrier()
    mvv = mv[...]
    asl = row_w - 16

    ei = (ei0, ei1, ei2, ei3)
    semI = (semI0, semI1, semI2, semI3)
    rowA = (rowA0, rowA1)
    rowB = (rowB0, rowB1)
    semA = (semA0, semA1)
    semB = (semB0, semB1)
    semS = (semS0, semS1)

    def idx_start(b, j):
        @pl.when(b < NBR)
        def _():
            pltpu.async_copy(ereal.at[0, w, b], ei[j].at[0], semI[j])
            pltpu.async_copy(ereal.at[1, w, b], ei[j].at[1], semI[j])

        @pl.when(b >= NBR)
        def _():
            pltpu.async_copy(edum.at[0, b - NBR], ei[j].at[0], semI[j])
            pltpu.async_copy(edum.at[1, b - NBR], ei[j].at[1], semI[j])

    def idx_wait(j):
        pltpu.make_async_copy(edum.at[0, 0], ei[j].at[0], semI[j]).wait()
        pltpu.make_async_copy(edum.at[1, 0], ei[j].at[1], semI[j]).wait()

    def gatherA(b, p, j):
        return pltpu.make_async_copy(tabA.at[ei[j].at[0]], rowA[p], semA[p])

    def gatherB(b, p, j):
        return pltpu.make_async_copy(tabB.at[ei[j].at[1]], rowB[p], semB[p])

    def scatter(b, p, j):
        return pltpu.make_async_copy(rowA[p], acc.at[ei[j].at[1]], semS[p])

    def compute(p):
        ra_ref = rowA[p]
        rb_ref = rowB[p]

        nh = (row_w - 16) // 16

        @plsc.parallel_loop(0, CB, step=1, unroll=4 if head_slots == 1 else 2)
        def edge(e):
            ra = ra_ref[e, pl.ds(asl, 16)]
            rb = rb_ref[e, pl.ds(0, 16)]
            hv = [ra_ref[e, pl.ds(k * 16, 16)] for k in range(nh)]
            sv = ra + rb
            al = jnp.where(sv > 0, sv, 0.2 * sv)
            p_ = jnp.exp(al - mvv)
            if head_slots == 8:
                ra_ref[e, pl.ds(asl, 16)] = p_
                for k in range(nh):
                    pk = _bcast_lane(p_, k)
                    ra_ref[e, pl.ds(k * 16, 16)] = hv[k] * pk
            else:
                pk = _bcast_lane(p_, 9)
                ra_ref[e, pl.ds(asl, 16)] = ra * pk
                for k in range(nh):
                    ra_ref[e, pl.ds(k * 16, 16)] = hv[k] * pk

    idx_start(0, 0)
    idx_start(1, 1)
    idx_wait(0)
    gatherA(0, 0, 0).start()
    gatherB(0, 0, 0).start()

    def quad(i, carry):
        for u in range(4):
            b = 4 * i + u
            p = u % 2
            j = u
            jn = (u + 1) % 4
            jp = (u + 2) % 4
            jq = (u + 3) % 4

            @pl.when(b >= 1)
            def _(b=b, p=p, jq=jq):
                scatter(b - 1, 1 - p, jq).wait()

            @pl.when(b + 2 < NBLK)
            def _(b=b, jp=jp):
                idx_start(b + 2, jp)

            @pl.when(b + 1 < NBLK)
            def _(b=b, p=p, jn=jn):
                idx_wait(jn)
                gatherA(b + 1, 1 - p, jn).start()
                gatherB(b + 1, 1 - p, jn).start()

            gatherA(b, p, j).wait()
            gatherB(b, p, j).wait()
            compute(p)
            pltpu.async_copy(rowA[p], acc.at[ei[j].at[1]], semS[p], add=True)
        return carry

    lax.fori_loop(0, NBLK // 4, quad, 0)
    scatter(NBLK - 1, 1, 3).wait()
    plsc.subcore_barrier()
    pltpu.sync_copy(acc.at[pl.ds(s * ROWS, ROWS)],
                    out.at[c, pl.ds(s * ROWS, ROWS)])


def _edge_pass(row_w, head_slots, tabA, tabB, ereal, edum, zrows, mvec):
    mesh = plsc.VectorSubcoreMesh(core_axis_name="c", subcore_axis_name="s",
                                  num_cores=NSC, num_subcores=NTEC)
    body = functools.partial(_edge_kernel_body, row_w, head_slots)
    return pl.kernel(
        body,
        out_type=jax.ShapeDtypeStruct((NSC, NPAD, row_w), _f32),
        mesh=mesh,
        scratch_types=[
            pltpu.VMEM((2, CB), jnp.int32),
            pltpu.VMEM((2, CB), jnp.int32),
            pltpu.VMEM((2, CB), jnp.int32),
            pltpu.VMEM((2, CB), jnp.int32),
            pltpu.VMEM((CB, row_w), _f32),
            pltpu.VMEM((CB, row_w), _f32),
            pltpu.VMEM((CB, 16), _f32),
            pltpu.VMEM((CB, 16), _f32),
            pltpu.VMEM((16,), _f32),
            pltpu.VMEM_SHARED((NPAD, row_w), _f32),
            pltpu.SemaphoreType.DMA,
            pltpu.SemaphoreType.DMA,
            pltpu.SemaphoreType.DMA,
            pltpu.SemaphoreType.DMA,
            pltpu.SemaphoreType.DMA,
            pltpu.SemaphoreType.DMA,
            pltpu.SemaphoreType.DMA,
            pltpu.SemaphoreType.DMA,
            pltpu.SemaphoreType.DMA,
            pltpu.SemaphoreType.DMA,
        ],
        compiler_params=pltpu.CompilerParams(use_tc_tiling_on_sc=False),
    )(tabA, tabB, ereal, edum, zrows, mvec)


def _s3_body(acc_ref, b1_ref, r_ref, w2_ref, s2_ref, d2_ref,
             tabA_ref, tabB_ref, mx_ref):
    i = pl.program_id(0)
    num = acc_ref[0] + acc_ref[1]
    den = num[:, D1:D1 + H1]
    dw = jnp.dot(den, r_ref[...], preferred_element_type=_f32)
    o1 = num[:, :D1] / (dw + 1e-16) + b1_ref[...]
    o1 = jnp.where(o1 > 0, o1, jnp.exp(o1) - 1.0)
    h2 = jnp.dot(o1, w2_ref[...], preferred_element_type=_f32)
    a2s = jnp.dot(h2, s2_ref[...], preferred_element_type=_f32)
    a2d = jnp.dot(h2, d2_ref[...], preferred_element_type=_f32)
    one = jnp.ones((BLK, 1), _f32)
    z6 = jnp.zeros((BLK, 6), _f32)
    z9 = jnp.zeros((BLK, 9), _f32)
    tabA_ref[...] = jnp.concatenate([h2, one, a2s, z6], axis=1)
    tabB_ref[...] = jnp.concatenate([z9, a2d, z6], axis=1)
    m = jnp.concatenate(
        [jnp.max(a2s, axis=0, keepdims=True),
         jnp.max(a2d, axis=0, keepdims=True),
         jnp.zeros((1, 14), _f32)], axis=1)

    @pl.when(i == 0)
    def _():
        mx_ref[...] = m

    @pl.when(i > 0)
    def _():
        mx_ref[...] = jnp.maximum(mx_ref[...], m)


def _stage3(acc1, b1, R, W2, s2, d2):
    return pl.pallas_call(
        _s3_body,
        grid=(GRID,),
        in_specs=[
            pl.BlockSpec((NSC, BLK, RA1), lambda i: (0, i, 0)),
            pl.BlockSpec((1, D1), lambda i: (0, 0)),
            pl.BlockSpec((H1, D1), lambda i: (0, 0)),
            pl.BlockSpec((D1, NCLS), lambda i: (0, 0)),
            pl.BlockSpec((NCLS, 1), lambda i: (0, 0)),
            pl.BlockSpec((NCLS, 1), lambda i: (0, 0)),
        ],
        out_specs=[
            pl.BlockSpec((BLK, RA2), lambda i: (i, 0)),
            pl.BlockSpec((BLK, 16), lambda i: (i, 0)),
            pl.BlockSpec((1, 16), lambda i: (0, 0)),
        ],
        out_shape=[
            jax.ShapeDtypeStruct((NPAD, RA2), _f32),
            jax.ShapeDtypeStruct((NPAD, 16), _f32),
            jax.ShapeDtypeStruct((1, 16), _f32),
        ],
    )(acc1, b1, R, W2, s2, d2)


def _s5_body(acc_ref, b2_ref, out_ref):
    num = acc_ref[0] + acc_ref[1]
    den = num[:, NCLS:NCLS + 1]
    lg = num[:, :NCLS] / (den + 1e-16) + b2_ref[...]
    m = jnp.max(lg, axis=1, keepdims=True)
    ls = lg - m
    out_ref[...] = ls - jnp.log(jnp.sum(jnp.exp(ls), axis=1, keepdims=True))


def _stage5(acc2, b2):
    return pl.pallas_call(
        _s5_body,
        grid=(GRID5,),
        in_specs=[
            pl.BlockSpec((NSC, BLK5, RA2), lambda i: (0, i, 0)),
            pl.BlockSpec((1, NCLS), lambda i: (0, 0)),
        ],
        out_specs=pl.BlockSpec((BLK5, NCLS), lambda i: (i, 0)),
        out_shape=jax.ShapeDtypeStruct((N, NCLS), _f32),
    )(acc2, b2)


def kernel(x, edge_index, W1, att_src1, att_dst1, b1, W2, att_src2, att_dst2,
           b2):
    edge_index = edge_index.astype(jnp.int32)
    ereal = edge_index.reshape(2, NW, NBR, CB)
    padrow = N + jnp.arange(NDUM * CB, dtype=jnp.int32) % (NPAD - N)
    edum = jnp.broadcast_to(padrow.reshape(NDUM, CB), (2, NDUM, CB))

    eye = jnp.eye(H1, dtype=_f32)
    As = (att_src1[:, :, None] * eye[:, None, :]).reshape(D1, H1)
    Ad = (att_dst1[:, :, None] * eye[:, None, :]).reshape(D1, H1)
    R = jnp.repeat(eye, F1, axis=1)

    tabA1, tabB1, mx1 = _stage1(x, W1, As, Ad)
    m1 = jnp.maximum(mx1[0, :H1] + mx1[0, H1:], 0.0)
    mvec1 = jnp.concatenate([m1, jnp.zeros((8,), _f32)])
    zrows1 = jnp.zeros((ROWS, RA1), _f32)

    acc1 = _edge_pass(RA1, H1, tabA1, tabB1, ereal, edum, zrows1, mvec1)

    tabA2, tabB2, mx2 = _stage3(acc1, b1.reshape(1, D1), R, W2,
                                att_src2.reshape(NCLS, 1),
                                att_dst2.reshape(NCLS, 1))
    m2 = jnp.maximum(mx2[0, 0] + mx2[0, 1], 0.0)
    mvec2 = jnp.zeros((16,), _f32).at[9].set(m2)
    zrows2 = jnp.zeros((ROWS, RA2), _f32)

    acc2 = _edge_pass(RA2, 1, tabA2, tabB2, ereal, edum, zrows2, mvec2)

    return _stage5(acc2, b2.reshape(1, NCLS))
